# baseline (speedup 1.0000x reference)
import functools

import jax
import jax.numpy as jnp
from jax import lax
from jax.experimental import pallas as pl
from jax.experimental.pallas import tpu as pltpu
from jax.experimental.pallas import tpu_sc as plsc

N_REL = 64
ED = 64
RD = 64
B = 4096
L2_LAMBDA = 1e-05

NW = 32
NG = 3 * B
GPW = NG // NW

BB = 512
NB = B // BB


def _sc_gather(entity_embed, idx_all):
    mesh = plsc.VectorSubcoreMesh(core_axis_name="c", subcore_axis_name="s")

    @functools.partial(
        pl.kernel,
        out_type=jax.ShapeDtypeStruct((NG, ED), jnp.float32),
        mesh=mesh,
        scratch_types=[
            pltpu.VMEM((GPW,), jnp.int32),
            pltpu.SemaphoreType.DMA,
        ],
    )
    def gather_k(ent_hbm, idx_hbm, out_hbm, idx_v, sem):
        wid = lax.axis_index("s") * 2 + lax.axis_index("c")
        base = wid * GPW
        pltpu.sync_copy(idx_hbm.at[pl.ds(base, GPW)], idx_v)

        def body(c, carry):
            vec = idx_v[pl.ds(c * 16, 16)]
            for lane in range(16):
                row = vec[lane]
                pltpu.async_copy(ent_hbm.at[pl.ds(row, 1), :],
                                 out_hbm.at[pl.ds(base + c * 16 + lane, 1), :],
                                 sem)
            return carry

        lax.fori_loop(0, GPW // 16, body, 0)
        pltpu.make_async_copy(ent_hbm.at[pl.ds(0, GPW), :],
                              out_hbm.at[pl.ds(base, GPW), :], sem).wait()

    return gather_k(entity_embed, idx_all)


def _tc_body(h_ref, p_ref, n_ref, rel_ref, r_ref, m_ref, out_ref, acc_ref):
    @pl.when(pl.program_id(0) == 0)
    def _init():
        acc_ref[0] = 0.0
        acc_ref[1] = 0.0

    w3 = m_ref[...].reshape(N_REL * ED, RD).astype(jnp.bfloat16)
    rcol = r_ref[...]
    lane_k = lax.broadcasted_iota(jnp.int32, (1, N_REL * ED), 1) // ED
    mask = lane_k == rcol
    zero = jnp.zeros((), jnp.bfloat16)

    def proj(x):
        xt = jnp.tile(x.astype(jnp.bfloat16), (1, N_REL))
        xm = jnp.where(mask, xt, zero)
        return lax.dot_general(xm, w3, (((1,), (0,)), ((), ())),
                               preferred_element_type=jnp.float32)

    rh = proj(h_ref[...])
    rp = proj(p_ref[...])
    rn = proj(n_ref[...])

    lane_r = lax.broadcasted_iota(jnp.int32, (1, N_REL), 1)
    oh = (lane_r == rcol).astype(jnp.float32)
    re = lax.dot_general(oh, rel_ref[...], (((1,), (0,)), ((), ())),
                         preferred_element_type=jnp.float32)

    anchor = rh + re
    pos_s = jnp.sum(jnp.square(anchor - rp), axis=1, keepdims=True)
    neg_s = jnp.sum(jnp.square(anchor - rn), axis=1, keepdims=True)
    d = neg_s - pos_s
    trip = jnp.maximum(-d, 0.0) + jnp.log(1.0 + jnp.exp(-jnp.abs(d)))
    l2 = 0.5 * (jnp.sum(jnp.square(rh)) + jnp.sum(jnp.square(re))
                + jnp.sum(jnp.square(rp)) + jnp.sum(jnp.square(rn)))

    acc_ref[0] += jnp.sum(trip)
    acc_ref[1] += l2

    @pl.when(pl.program_id(0) == NB - 1)
    def _fin():
        loss = acc_ref[0] / B + L2_LAMBDA * (acc_ref[1] / B)
        out_ref[...] = jnp.full((1, 1), loss, dtype=jnp.float32)


def _tc_loss(ent_rows, relation_embed, r2, trans_M):
    ent_spec = lambda a: pl.BlockSpec((BB, ED), lambda i, a=a: (i + a * NB, 0))
    return pl.pallas_call(
        _tc_body,
        grid=(NB,),
        in_specs=[
            ent_spec(0),
            ent_spec(1),
            ent_spec(2),
            pl.BlockSpec((N_REL, RD), lambda i: (0, 0)),
            pl.BlockSpec((BB, 1), lambda i: (i, 0)),
            pl.BlockSpec((N_REL, ED, RD), lambda i: (0, 0, 0)),
        ],
        out_specs=pl.BlockSpec((1, 1), lambda i: (0, 0)),
        out_shape=jax.ShapeDtypeStruct((1, 1), jnp.float32),
        scratch_shapes=[pltpu.SMEM((2,), jnp.float32)],
        compiler_params=pltpu.CompilerParams(
            dimension_semantics=("arbitrary",)),
    )(ent_rows, ent_rows, ent_rows, relation_embed, r2, trans_M)


def kernel(h, r, pos_t, neg_t, entity_embed, relation_embed, trans_M):
    h = h.astype(jnp.int32)
    r = r.astype(jnp.int32)
    pos_t = pos_t.astype(jnp.int32)
    neg_t = neg_t.astype(jnp.int32)
    idx_all = jnp.concatenate([h, pos_t, neg_t])
    ent_rows = _sc_gather(entity_embed, idx_all)
    out = _tc_loss(ent_rows, relation_embed, r.reshape(B, 1), trans_M)
    return out[0, 0]


# speedup vs baseline: 2.1833x; 2.1833x over previous
import functools

import jax
import jax.numpy as jnp
from jax import lax
from jax.experimental import pallas as pl
from jax.experimental.pallas import tpu as pltpu
from jax.experimental.pallas import tpu_sc as plsc

N_REL = 64
ED = 64
RD = 64
B = 4096
L2_LAMBDA = 1e-05

NW = 32
NG = 3 * B
GPW = NG // NW
GC = 128

BB = 1024
NB = B // BB


def _sc_gather(ent_pairs, idx_half):
    mesh = plsc.VectorSubcoreMesh(core_axis_name="c", subcore_axis_name="s")

    @functools.partial(
        pl.kernel,
        out_type=jax.ShapeDtypeStruct((NG, 2 * ED), jnp.float32),
        mesh=mesh,
        scratch_types=[
            pltpu.VMEM((GPW,), jnp.int32),
            pltpu.VMEM((GC, 2 * ED), jnp.float32),
            pltpu.SemaphoreType.DMA,
        ],
    )
    def gather_k(ent_hbm, idx_hbm, out_hbm, idx_v, rows_v, sem):
        wid = lax.axis_index("s") * 2 + lax.axis_index("c")
        base = wid * GPW
        pltpu.sync_copy(idx_hbm.at[pl.ds(base, GPW)], idx_v)
        for c in range(GPW // GC):
            pltpu.async_copy(ent_hbm.at[idx_v.at[pl.ds(c * GC, GC)]],
                             rows_v, sem).wait()
            pltpu.sync_copy(rows_v, out_hbm.at[pl.ds(base + c * GC, GC)])

    return gather_k(ent_pairs, idx_half)


def _tc_body(h_ref, p_ref, n_ref, hp_ref, pp_ref, np_ref, rel_ref, r_ref,
             w3_ref, out_ref, acc_ref):
    @pl.when(pl.program_id(0) == 0)
    def _init():
        acc_ref[0] = 0.0
        acc_ref[1] = 0.0

    w3 = w3_ref[...]
    rcol = r_ref[...]
    lane_k = lax.broadcasted_iota(jnp.int32, (1, N_REL * ED), 1) // ED
    mask = lane_k == rcol
    zero = jnp.zeros((), jnp.bfloat16)

    def proj(pair_ref, par_ref):
        pair = pair_ref[...]
        par = par_ref[...]
        x = jnp.where(par == 1, pair[:, ED:2 * ED], pair[:, 0:ED])
        xt = jnp.tile(x.astype(jnp.bfloat16), (1, N_REL))
        xm = jnp.where(mask, xt, zero)
        return lax.dot_general(xm, w3, (((1,), (0,)), ((), ())),
                               preferred_element_type=jnp.float32)

    rh = proj(h_ref, hp_ref)
    rp = proj(p_ref, pp_ref)
    rn = proj(n_ref, np_ref)

    lane_r = lax.broadcasted_iota(jnp.int32, (1, N_REL), 1)
    oh = (lane_r == rcol).astype(jnp.float32)
    re = lax.dot_general(oh, rel_ref[...], (((1,), (0,)), ((), ())),
                         preferred_element_type=jnp.float32)

    anchor = rh + re
    pos_s = jnp.sum(jnp.square(anchor - rp), axis=1, keepdims=True)
    neg_s = jnp.sum(jnp.square(anchor - rn), axis=1, keepdims=True)
    d = neg_s - pos_s
    trip = jnp.maximum(-d, 0.0) + jnp.log(1.0 + jnp.exp(-jnp.abs(d)))
    l2 = 0.5 * (jnp.sum(jnp.square(rh)) + jnp.sum(jnp.square(re))
                + jnp.sum(jnp.square(rp)) + jnp.sum(jnp.square(rn)))

    acc_ref[0] += jnp.sum(trip)
    acc_ref[1] += l2

    @pl.when(pl.program_id(0) == NB - 1)
    def _fin():
        loss = acc_ref[0] / B + L2_LAMBDA * (acc_ref[1] / B)
        out_ref[...] = jnp.full((1, 1), loss, dtype=jnp.float32)


def _tc_loss(ent_pairs_rows, parity, relation_embed, r2, w3_16):
    ent_spec = lambda a: pl.BlockSpec((BB, 2 * ED), lambda i, a=a: (i + a * NB, 0))
    par_spec = lambda a: pl.BlockSpec((BB, 1), lambda i, a=a: (i + a * NB, 0))
    return pl.pallas_call(
        _tc_body,
        grid=(NB,),
        in_specs=[
            ent_spec(0), ent_spec(1), ent_spec(2),
            par_spec(0), par_spec(1), par_spec(2),
            pl.BlockSpec((N_REL, RD), lambda i: (0, 0)),
            pl.BlockSpec((BB, 1), lambda i: (i, 0)),
            pl.BlockSpec((N_REL * ED, RD), lambda i: (0, 0)),
        ],
        out_specs=pl.BlockSpec((1, 1), lambda i: (0, 0)),
        out_shape=jax.ShapeDtypeStruct((1, 1), jnp.float32),
        scratch_shapes=[pltpu.SMEM((2,), jnp.float32)],
        compiler_params=pltpu.CompilerParams(
            dimension_semantics=("arbitrary",)),
    )(ent_pairs_rows, ent_pairs_rows, ent_pairs_rows,
      parity, parity, parity, relation_embed, r2, w3_16)


def kernel(h, r, pos_t, neg_t, entity_embed, relation_embed, trans_M):
    h = h.astype(jnp.int32)
    r = r.astype(jnp.int32)
    pos_t = pos_t.astype(jnp.int32)
    neg_t = neg_t.astype(jnp.int32)
    idx_all = jnp.concatenate([h, pos_t, neg_t])
    ent_pairs = entity_embed.reshape(50000, 2 * ED)
    rows = _sc_gather(ent_pairs, idx_all >> 1)
    parity = (idx_all & 1).reshape(NG, 1)
    w3_16 = trans_M.reshape(N_REL * ED, RD).astype(jnp.bfloat16)
    out = _tc_loss(rows, parity, relation_embed, r.reshape(B, 1), w3_16)
    return out[0, 0]


# speedup vs baseline: 2.2924x; 1.0500x over previous
import functools

import jax
import jax.numpy as jnp
from jax import lax
from jax.experimental import pallas as pl
from jax.experimental.pallas import tpu as pltpu
from jax.experimental.pallas import tpu_sc as plsc

N_REL = 64
ED = 64
RD = 64
B = 4096
L2_LAMBDA = 1e-05

NW = 32
NG = 3 * B
GPW = NG // NW
GC = 128

BB = 1024
NB = B // BB

HALF = 50000
DP_CH = 10000


def _depad_body(a_ref, b_ref, o_ref):
    o_ref[...] = jnp.concatenate([a_ref[...], b_ref[...]], axis=1)


def _depad(entity_embed):
    nsteps = HALF // DP_CH
    return pl.pallas_call(
        _depad_body,
        grid=(nsteps,),
        in_specs=[
            pl.BlockSpec((DP_CH, ED), lambda i: (i, 0)),
            pl.BlockSpec((DP_CH, ED), lambda i, n=nsteps: (i + n, 0)),
        ],
        out_specs=pl.BlockSpec((DP_CH, 2 * ED), lambda i: (i, 0)),
        out_shape=jax.ShapeDtypeStruct((HALF, 2 * ED), jnp.float32),
    )(entity_embed, entity_embed)


def _sc_gather(ent_pairs, idx_half):
    mesh = plsc.VectorSubcoreMesh(core_axis_name="c", subcore_axis_name="s")

    @functools.partial(
        pl.kernel,
        out_type=jax.ShapeDtypeStruct((NG, 2 * ED), jnp.float32),
        mesh=mesh,
        scratch_types=[
            pltpu.VMEM((GPW,), jnp.int32),
            pltpu.VMEM((GC, 2 * ED), jnp.float32),
            pltpu.SemaphoreType.DMA,
        ],
        compiler_params=pltpu.CompilerParams(use_tc_tiling_on_sc=True),
    )
    def gather_k(ent_hbm, idx_hbm, out_hbm, idx_v, rows_v, sem):
        wid = lax.axis_index("s") * 2 + lax.axis_index("c")
        base = wid * GPW
        pltpu.sync_copy(idx_hbm.at[pl.ds(base, GPW)], idx_v)
        for c in range(GPW // GC):
            pltpu.async_copy(ent_hbm.at[idx_v.at[pl.ds(c * GC, GC)]],
                             rows_v, sem).wait()
            pltpu.sync_copy(rows_v, out_hbm.at[pl.ds(base + c * GC, GC)])

    return gather_k(ent_pairs, idx_half)


def _tc_body(h_ref, p_ref, n_ref, hp_ref, pp_ref, np_ref, rel_ref, r_ref,
             w3_ref, out_ref, acc_ref):
    @pl.when(pl.program_id(0) == 0)
    def _init():
        acc_ref[0] = 0.0
        acc_ref[1] = 0.0

    w3 = w3_ref[...]
    rcol = jnp.transpose(r_ref[0], (1, 0))
    lane_k = lax.broadcasted_iota(jnp.int32, (1, N_REL * ED), 1) // ED
    mask = lane_k == rcol
    zero = jnp.zeros((), jnp.bfloat16)

    def proj(pair_ref, par_ref):
        pair = pair_ref[...]
        par = jnp.transpose(par_ref[0], (1, 0))
        x = jnp.where(par == 1, pair[:, ED:2 * ED], pair[:, 0:ED])
        xt = jnp.tile(x.astype(jnp.bfloat16), (1, N_REL))
        xm = jnp.where(mask, xt, zero)
        return lax.dot_general(xm, w3, (((1,), (0,)), ((), ())),
                               preferred_element_type=jnp.float32)

    rh = proj(h_ref, hp_ref)
    rp = proj(p_ref, pp_ref)
    rn = proj(n_ref, np_ref)

    lane_r = lax.broadcasted_iota(jnp.int32, (1, N_REL), 1)
    oh = (lane_r == rcol).astype(jnp.float32)
    re = lax.dot_general(oh, rel_ref[...], (((1,), (0,)), ((), ())),
                         preferred_element_type=jnp.float32)

    anchor = rh + re
    pos_s = jnp.sum(jnp.square(anchor - rp), axis=1, keepdims=True)
    neg_s = jnp.sum(jnp.square(anchor - rn), axis=1, keepdims=True)
    d = neg_s - pos_s
    trip = jnp.maximum(-d, 0.0) + jnp.log(1.0 + jnp.exp(-jnp.abs(d)))
    l2 = 0.5 * (jnp.sum(jnp.square(rh)) + jnp.sum(jnp.square(re))
                + jnp.sum(jnp.square(rp)) + jnp.sum(jnp.square(rn)))

    acc_ref[0] += jnp.sum(trip)
    acc_ref[1] += l2

    @pl.when(pl.program_id(0) == NB - 1)
    def _fin():
        loss = acc_ref[0] / B + L2_LAMBDA * (acc_ref[1] / B)
        out_ref[...] = jnp.full((1, 1), loss, dtype=jnp.float32)


def _tc_loss(ent_pairs_rows, parity, relation_embed, r_rows, w3_16):
    ent_spec = lambda a: pl.BlockSpec((BB, 2 * ED), lambda i, a=a: (i + a * NB, 0))
    par_spec = lambda a: pl.BlockSpec((1, 1, BB), lambda i, a=a: (a * NB + i, 0, 0))
    return pl.pallas_call(
        _tc_body,
        grid=(NB,),
        in_specs=[
            ent_spec(0), ent_spec(1), ent_spec(2),
            par_spec(0), par_spec(1), par_spec(2),
            pl.BlockSpec((N_REL, RD), lambda i: (0, 0)),
            pl.BlockSpec((1, 1, BB), lambda i: (i, 0, 0)),
            pl.BlockSpec((N_REL * ED, RD), lambda i: (0, 0)),
        ],
        out_specs=pl.BlockSpec((1, 1), lambda i: (0, 0)),
        out_shape=jax.ShapeDtypeStruct((1, 1), jnp.float32),
        scratch_shapes=[pltpu.SMEM((2,), jnp.float32)],
        compiler_params=pltpu.CompilerParams(
            dimension_semantics=("arbitrary",)),
    )(ent_pairs_rows, ent_pairs_rows, ent_pairs_rows,
      parity, parity, parity, relation_embed, r_rows, w3_16)


def kernel(h, r, pos_t, neg_t, entity_embed, relation_embed, trans_M):
    h = h.astype(jnp.int32)
    r = r.astype(jnp.int32)
    pos_t = pos_t.astype(jnp.int32)
    neg_t = neg_t.astype(jnp.int32)
    idx_all = jnp.concatenate([h, pos_t, neg_t])
    ent_pairs = _depad(entity_embed)
    hi = idx_all >= HALF
    rows = _sc_gather(ent_pairs, jnp.where(hi, idx_all - HALF, idx_all))
    parity = hi.astype(jnp.int32).reshape(3 * NB, 1, BB)
    w3_16 = trans_M.reshape(N_REL * ED, RD).astype(jnp.bfloat16)
    out = _tc_loss(rows, parity, relation_embed,
                   r.reshape(NB, 1, BB), w3_16)
    return out[0, 0]


# speedup vs baseline: 2.6002x; 1.1343x over previous
import functools

import jax
import jax.numpy as jnp
from jax import lax
from jax.experimental import pallas as pl
from jax.experimental.pallas import tpu as pltpu
from jax.experimental.pallas import tpu_sc as plsc

N_REL = 64
ED = 64
RD = 64
B = 4096
L2_LAMBDA = 1e-05

G = 8
GE = G * ED

NW = 32
NG = 3 * B
GPW = NG // NW
GC = 128

BB = 2048
NB = B // BB

HALF = 50000
DP = 5000


def _depad_body(x_ref, o_ref):
    x = x_ref[...]
    o_ref[...] = jnp.concatenate([x[:DP, :], x[DP:, :]], axis=1)


def _depad(entity_embed):
    return pl.pallas_call(
        _depad_body,
        grid=(HALF // DP,),
        in_specs=[pl.BlockSpec((2 * DP, ED), lambda i: (i, 0))],
        out_specs=pl.BlockSpec((DP, 2 * ED), lambda i: (i, 0)),
        out_shape=jax.ShapeDtypeStruct((HALF, 2 * ED), jnp.float32),
    )(entity_embed)


def _sc_gather(ent_pairs, idx_packed):
    mesh = plsc.VectorSubcoreMesh(core_axis_name="c", subcore_axis_name="s")

    @functools.partial(
        pl.kernel,
        out_type=jax.ShapeDtypeStruct((NG, 2 * ED), jnp.float32),
        mesh=mesh,
        scratch_types=[
            pltpu.VMEM((GPW,), jnp.int32),
            pltpu.VMEM((GC, 2 * ED), jnp.float32),
            pltpu.SemaphoreType.DMA,
        ],
        compiler_params=pltpu.CompilerParams(use_tc_tiling_on_sc=True),
    )
    def gather_k(ent_hbm, idx_hbm, out_hbm, idx_v, rows_v, sem):
        wid = lax.axis_index("s") * 2 + lax.axis_index("c")
        base = wid * GPW
        pltpu.sync_copy(idx_hbm.at[pl.ds(base, GPW)], idx_v)
        for c in range(GPW // GC):
            pltpu.async_copy(ent_hbm.at[idx_v.at[pl.ds(c * GC, GC)]],
                             rows_v, sem).wait()
            pltpu.sync_copy(rows_v, out_hbm.at[pl.ds(base + c * GC, GC)])

    return gather_k(ent_pairs, idx_packed)


def _tc_body(h_ref, p_ref, n_ref, hp_ref, pp_ref, np_ref, rel_ref, r_ref,
             wg_ref, out_ref, acc_ref):
    @pl.when(pl.program_id(0) == 0)
    def _init():
        acc_ref[0] = 0.0
        acc_ref[1] = 0.0

    wg = wg_ref[...]
    rcol = jnp.transpose(r_ref[0], (1, 0))
    r1 = rcol // G
    r0 = rcol % G
    lane_g = lax.broadcasted_iota(jnp.int32, (1, GE), 1) // ED
    mask1 = lane_g == r1
    mask0 = lane_g == r0
    zero16 = jnp.zeros((), jnp.bfloat16)
    f0 = (lax.broadcasted_iota(jnp.int32, (GE, RD), 0) % ED
          == lax.broadcasted_iota(jnp.int32, (GE, RD), 1)).astype(jnp.float32)

    def proj(pair_ref, par_ref):
        pair = pair_ref[...]
        par = jnp.transpose(par_ref[0], (1, 0))
        x = jnp.where(par == 1, pair[:, ED:2 * ED], pair[:, 0:ED])
        xt = jnp.tile(x.astype(jnp.bfloat16), (1, G))
        x1 = jnp.where(mask1, xt, zero16)
        y = lax.dot_general(x1, wg, (((1,), (0,)), ((), ())),
                            preferred_element_type=jnp.float32)
        ys = jnp.where(mask0, y, 0.0)
        return lax.dot_general(ys, f0, (((1,), (0,)), ((), ())),
                               preferred_element_type=jnp.float32)

    rh = proj(h_ref, hp_ref)
    rp = proj(p_ref, pp_ref)
    rn = proj(n_ref, np_ref)

    lane_r = lax.broadcasted_iota(jnp.int32, (1, N_REL), 1)
    oh = (lane_r == rcol).astype(jnp.float32)
    re = lax.dot_general(oh, rel_ref[...], (((1,), (0,)), ((), ())),
                         preferred_element_type=jnp.float32)

    anchor = rh + re
    pos_s = jnp.sum(jnp.square(anchor - rp), axis=1, keepdims=True)
    neg_s = jnp.sum(jnp.square(anchor - rn), axis=1, keepdims=True)
    d = neg_s - pos_s
    trip = jnp.maximum(-d, 0.0) + jnp.log(1.0 + jnp.exp(-jnp.abs(d)))
    l2 = 0.5 * (jnp.sum(jnp.square(rh)) + jnp.sum(jnp.square(re))
                + jnp.sum(jnp.square(rp)) + jnp.sum(jnp.square(rn)))

    acc_ref[0] += jnp.sum(trip)
    acc_ref[1] += l2

    @pl.when(pl.program_id(0) == NB - 1)
    def _fin():
        loss = acc_ref[0] / B + L2_LAMBDA * (acc_ref[1] / B)
        out_ref[...] = jnp.full((1, 1), loss, dtype=jnp.float32)


def _tc_loss(ent_pairs_rows, parity, relation_embed, r_rows, wg_16):
    ent_spec = lambda a: pl.BlockSpec((BB, 2 * ED), lambda i, a=a: (i + a * NB, 0))
    par_spec = lambda a: pl.BlockSpec((1, 1, BB), lambda i, a=a: (a * NB + i, 0, 0))
    return pl.pallas_call(
        _tc_body,
        grid=(NB,),
        in_specs=[
            ent_spec(0), ent_spec(1), ent_spec(2),
            par_spec(0), par_spec(1), par_spec(2),
            pl.BlockSpec((N_REL, RD), lambda i: (0, 0)),
            pl.BlockSpec((1, 1, BB), lambda i: (i, 0, 0)),
            pl.BlockSpec((GE, GE), lambda i: (0, 0)),
        ],
        out_specs=pl.BlockSpec((1, 1), lambda i: (0, 0)),
        out_shape=jax.ShapeDtypeStruct((1, 1), jnp.float32),
        scratch_shapes=[pltpu.SMEM((2,), jnp.float32)],
        compiler_params=pltpu.CompilerParams(
            dimension_semantics=("arbitrary",)),
    )(ent_pairs_rows, ent_pairs_rows, ent_pairs_rows,
      parity, parity, parity, relation_embed, r_rows, wg_16)


def _pack_index(idx):
    step = idx // (2 * DP)
    rin = idx % (2 * DP)
    par = (rin >= DP).astype(jnp.int32)
    packed = step * DP + rin - par * DP
    return packed, par


def kernel(h, r, pos_t, neg_t, entity_embed, relation_embed, trans_M):
    h = h.astype(jnp.int32)
    r = r.astype(jnp.int32)
    pos_t = pos_t.astype(jnp.int32)
    neg_t = neg_t.astype(jnp.int32)
    idx_all = jnp.concatenate([h, pos_t, neg_t])
    ent_pairs = _depad(entity_embed)
    idx_packed, par = _pack_index(idx_all)
    rows = _sc_gather(ent_pairs, idx_packed)
    parity = par.reshape(3 * NB, 1, BB)
    wg_16 = (trans_M.reshape(G, G, ED, RD).transpose(0, 2, 1, 3)
             .reshape(GE, GE).astype(jnp.bfloat16))
    out = _tc_loss(rows, parity, relation_embed,
                   r.reshape(NB, 1, BB), wg_16)
    return out[0, 0]


# speedup vs baseline: 3.8936x; 1.4974x over previous
import functools

import jax
import jax.numpy as jnp
from jax import lax
from jax.experimental import pallas as pl
from jax.experimental.pallas import tpu as pltpu
from jax.experimental.pallas import tpu_sc as plsc

N_REL = 64
ED = 64
RD = 64
B = 4096
L2_LAMBDA = 1e-05

G = 8
GE = G * ED

NW = 32
NG = 3 * B
GPW = NG // NW
GC = 128

BB = 2048
NB = B // BB

PACKED = 51200
DP = 6400


def _depad_body(x_ref, o_ref):
    x = jnp.transpose(x_ref[...], (1, 0))
    o_ref[...] = jnp.concatenate([x[:DP, :], x[DP:, :]], axis=1)


def _depad(entity_t):
    return pl.pallas_call(
        _depad_body,
        grid=(PACKED // DP,),
        in_specs=[pl.BlockSpec((ED, 2 * DP), lambda i: (0, i))],
        out_specs=pl.BlockSpec((DP, 2 * ED), lambda i: (i, 0)),
        out_shape=jax.ShapeDtypeStruct((PACKED, 2 * ED), jnp.float32),
    )(entity_t)


def _sc_gather(ent_pairs, idx_packed):
    mesh = plsc.VectorSubcoreMesh(core_axis_name="c", subcore_axis_name="s")

    @functools.partial(
        pl.kernel,
        out_type=jax.ShapeDtypeStruct((NG, 2 * ED), jnp.float32),
        mesh=mesh,
        scratch_types=[
            pltpu.VMEM((GPW,), jnp.int32),
            pltpu.VMEM((GC, 2 * ED), jnp.float32),
            pltpu.SemaphoreType.DMA,
        ],
        compiler_params=pltpu.CompilerParams(use_tc_tiling_on_sc=True),
    )
    def gather_k(ent_hbm, idx_hbm, out_hbm, idx_v, rows_v, sem):
        wid = lax.axis_index("s") * 2 + lax.axis_index("c")
        base = wid * GPW
        pltpu.sync_copy(idx_hbm.at[pl.ds(base, GPW)], idx_v)
        for c in range(GPW // GC):
            pltpu.async_copy(ent_hbm.at[idx_v.at[pl.ds(c * GC, GC)]],
                             rows_v, sem).wait()
            pltpu.sync_copy(rows_v, out_hbm.at[pl.ds(base + c * GC, GC)])

    return gather_k(ent_pairs, idx_packed)


def _tc_body(h_ref, p_ref, n_ref, hp_ref, pp_ref, np_ref, rel_ref, r_ref,
             wg_ref, out_ref, acc_ref):
    @pl.when(pl.program_id(0) == 0)
    def _init():
        acc_ref[0] = 0.0
        acc_ref[1] = 0.0

    wg = wg_ref[...]
    rcol = jnp.transpose(r_ref[0], (1, 0))
    r1 = rcol // G
    r0 = rcol % G
    lane_g = lax.broadcasted_iota(jnp.int32, (1, GE), 1) // ED
    mask1 = lane_g == r1
    mask0 = lane_g == r0
    zero16 = jnp.zeros((), jnp.bfloat16)
    f0 = (lax.broadcasted_iota(jnp.int32, (GE, RD), 0) % ED
          == lax.broadcasted_iota(jnp.int32, (GE, RD), 1)).astype(jnp.float32)

    def proj(pair_ref, par_ref):
        pair = pair_ref[...]
        par = jnp.transpose(par_ref[0], (1, 0))
        x = jnp.where(par == 1, pair[:, ED:2 * ED], pair[:, 0:ED])
        xt = jnp.tile(x.astype(jnp.bfloat16), (1, G))
        x1 = jnp.where(mask1, xt, zero16)
        y = lax.dot_general(x1, wg, (((1,), (0,)), ((), ())),
                            preferred_element_type=jnp.float32)
        ys = jnp.where(mask0, y, 0.0)
        return lax.dot_general(ys, f0, (((1,), (0,)), ((), ())),
                               preferred_element_type=jnp.float32)

    rh = proj(h_ref, hp_ref)
    rp = proj(p_ref, pp_ref)
    rn = proj(n_ref, np_ref)

    lane_r = lax.broadcasted_iota(jnp.int32, (1, N_REL), 1)
    oh = (lane_r == rcol).astype(jnp.float32)
    re = lax.dot_general(oh, rel_ref[...], (((1,), (0,)), ((), ())),
                         preferred_element_type=jnp.float32)

    anchor = rh + re
    pos_s = jnp.sum(jnp.square(anchor - rp), axis=1, keepdims=True)
    neg_s = jnp.sum(jnp.square(anchor - rn), axis=1, keepdims=True)
    d = neg_s - pos_s
    trip = jnp.maximum(-d, 0.0) + jnp.log(1.0 + jnp.exp(-jnp.abs(d)))
    l2 = 0.5 * (jnp.sum(jnp.square(rh)) + jnp.sum(jnp.square(re))
                + jnp.sum(jnp.square(rp)) + jnp.sum(jnp.square(rn)))

    acc_ref[0] += jnp.sum(trip)
    acc_ref[1] += l2

    @pl.when(pl.program_id(0) == NB - 1)
    def _fin():
        loss = acc_ref[0] / B + L2_LAMBDA * (acc_ref[1] / B)
        out_ref[...] = jnp.full((1, 1), loss, dtype=jnp.float32)


def _tc_loss(ent_pairs_rows, parity, relation_embed, r_rows, wg_16):
    ent_spec = lambda a: pl.BlockSpec((BB, 2 * ED), lambda i, a=a: (i + a * NB, 0))
    par_spec = lambda a: pl.BlockSpec((1, 1, BB), lambda i, a=a: (a * NB + i, 0, 0))
    return pl.pallas_call(
        _tc_body,
        grid=(NB,),
        in_specs=[
            ent_spec(0), ent_spec(1), ent_spec(2),
            par_spec(0), par_spec(1), par_spec(2),
            pl.BlockSpec((N_REL, RD), lambda i: (0, 0)),
            pl.BlockSpec((1, 1, BB), lambda i: (i, 0, 0)),
            pl.BlockSpec((GE, GE), lambda i: (0, 0)),
        ],
        out_specs=pl.BlockSpec((1, 1), lambda i: (0, 0)),
        out_shape=jax.ShapeDtypeStruct((1, 1), jnp.float32),
        scratch_shapes=[pltpu.SMEM((2,), jnp.float32)],
        compiler_params=pltpu.CompilerParams(
            dimension_semantics=("arbitrary",)),
    )(ent_pairs_rows, ent_pairs_rows, ent_pairs_rows,
      parity, parity, parity, relation_embed, r_rows, wg_16)


def _pack_index(idx):
    step = idx // (2 * DP)
    rin = idx % (2 * DP)
    par = (rin >= DP).astype(jnp.int32)
    packed = step * DP + rin - par * DP
    return packed, par


def kernel(h, r, pos_t, neg_t, entity_embed, relation_embed, trans_M):
    h = h.astype(jnp.int32)
    r = r.astype(jnp.int32)
    pos_t = pos_t.astype(jnp.int32)
    neg_t = neg_t.astype(jnp.int32)
    idx_all = jnp.concatenate([h, pos_t, neg_t])
    ent_pairs = _depad(entity_embed.T)
    idx_packed, par = _pack_index(idx_all)
    rows = _sc_gather(ent_pairs, idx_packed)
    parity = par.reshape(3 * NB, 1, BB)
    wg_16 = (trans_M.reshape(G, G, ED, RD).transpose(0, 2, 1, 3)
             .reshape(GE, GE).astype(jnp.bfloat16))
    out = _tc_loss(rows, parity, relation_embed,
                   r.reshape(NB, 1, BB), wg_16)
    return out[0, 0]
